# baseline (speedup 1.0000x reference)
import functools
import math

import numpy as np
import jax
import jax.numpy as jnp
from jax import lax
from jax.experimental import pallas as pl
from jax.experimental.pallas import tpu as pltpu
from jax.experimental.pallas import tpu_sc as plsc

_B, _S, _E = 4096, 200, 64
_NW = 32
_ROWS_PER_W = _B // _NW


def _pos_encoding_np(seq_len=_S, emb_size=_E):
    position_idx = np.arange(0, seq_len, dtype=np.float32)[:, None]
    fill = position_idx * np.exp(
        -np.arange(0, emb_size, 2, dtype=np.float32) * math.log(10000.0) / emb_size)
    pos = np.zeros((seq_len, emb_size), dtype=np.float32)
    pos[:, 0::2] = np.sin(fill)
    pos[:, 1::2] = np.cos(fill)
    return pos


_POS = _pos_encoding_np()


def _make_kernel():
    mesh = plsc.VectorSubcoreMesh(core_axis_name="c", subcore_axis_name="s")

    @functools.partial(
        pl.kernel,
        out_type=jax.ShapeDtypeStruct((_B, _S, _E), jnp.float32),
        mesh=mesh,
        scratch_types=[
            pltpu.VMEM((_S,), jnp.int32),
            pltpu.VMEM((_S, _E), jnp.float32),
            pltpu.VMEM((_S, _E), jnp.float32),
            pltpu.SemaphoreType.DMA,
        ],
        compiler_params=pltpu.CompilerParams(use_tc_tiling_on_sc=False),
    )
    def emb_kernel(x_hbm, table_hbm, pos_hbm, out_hbm, idx_v, rows_v, pos_v, sem):
        wid = lax.axis_index("s") * 2 + lax.axis_index("c")
        base = wid * _ROWS_PER_W

        pltpu.sync_copy(pos_hbm, pos_v)

        def body(i, carry):
            b = base + i
            pltpu.sync_copy(x_hbm.at[b], idx_v)
            pltpu.async_copy(table_hbm.at[idx_v], rows_v, sem).wait()

            def row_body(r, c2):
                for c in range(_E // 16):
                    sl = pl.ds(c * 16, 16)
                    rows_v[r, sl] = rows_v[r, sl] + pos_v[r, sl]
                return c2

            lax.fori_loop(0, _S, row_body, 0)
            pltpu.sync_copy(rows_v, out_hbm.at[b])
            return carry

        lax.fori_loop(0, _ROWS_PER_W, body, 0)

    return emb_kernel


_EMB_KERNEL = _make_kernel()


@jax.jit
def kernel(x, emb_table):
    pos = jnp.asarray(_POS)
    return _EMB_KERNEL(x.astype(jnp.int32), emb_table, pos)


# speedup vs baseline: 1.1890x; 1.1890x over previous
import functools
import math

import numpy as np
import jax
import jax.numpy as jnp
from jax import lax
from jax.experimental import pallas as pl
from jax.experimental.pallas import tpu as pltpu
from jax.experimental.pallas import tpu_sc as plsc

_B, _S, _E = 4096, 200, 64
_NW = 32
_ROWS_PER_W = _B // _NW
_NBUF = 4
_AHEAD = 2


def _pos_encoding_np(seq_len=_S, emb_size=_E):
    position_idx = np.arange(0, seq_len, dtype=np.float32)[:, None]
    fill = position_idx * np.exp(
        -np.arange(0, emb_size, 2, dtype=np.float32) * math.log(10000.0) / emb_size)
    pos = np.zeros((seq_len, emb_size), dtype=np.float32)
    pos[:, 0::2] = np.sin(fill)
    pos[:, 1::2] = np.cos(fill)
    return pos


_POS = _pos_encoding_np()


def _make_kernel():
    mesh = plsc.VectorSubcoreMesh(core_axis_name="c", subcore_axis_name="s")

    row_buf = pltpu.VMEM((_S, _E), jnp.float32)
    scratch = (
        [pltpu.VMEM((_ROWS_PER_W, _S), jnp.int32)]
        + [pltpu.VMEM((_S, _E), jnp.float32)]
        + [row_buf] * _NBUF
        + [pltpu.SemaphoreType.DMA] * _NBUF
        + [pltpu.SemaphoreType.DMA] * _NBUF
    )

    @functools.partial(
        pl.kernel,
        out_type=jax.ShapeDtypeStruct((_B, _S, _E), jnp.float32),
        mesh=mesh,
        scratch_types=scratch,
        compiler_params=pltpu.CompilerParams(use_tc_tiling_on_sc=False),
    )
    def emb_kernel(x_hbm, table_hbm, pos_hbm, out_hbm, idx_all, pos_v,
                   r0, r1, r2, r3, g0, g1, g2, g3, w0, w1, w2, w3):
        rows = (r0, r1, r2, r3)
        gsem = (g0, g1, g2, g3)
        wsem = (w0, w1, w2, w3)

        wid = lax.axis_index("s") * 2 + lax.axis_index("c")
        base = wid * _ROWS_PER_W

        pltpu.sync_copy(pos_hbm, pos_v)
        pltpu.sync_copy(x_hbm.at[pl.ds(base, _ROWS_PER_W)], idx_all)

        def start_gather(v, b):
            pltpu.async_copy(table_hbm.at[idx_all.at[v]], rows[b], gsem[b])

        def wait_gather(b):
            pltpu.make_async_copy(table_hbm.at[idx_all.at[0]], rows[b],
                                  gsem[b]).wait()

        def wait_write(b):
            pltpu.make_async_copy(rows[b], out_hbm.at[base], wsem[b]).wait()

        def add_pos(b):
            buf = rows[b]

            @plsc.parallel_loop(0, _S, unroll=4)
            def _(r):
                for c in range(_E // 16):
                    sl = pl.ds(c * 16, 16)
                    buf[r, sl] = buf[r, sl] + pos_v[r, sl]

        def visit(v, b, do_wwait, do_gather):
            wait_gather(b)
            add_pos(b)
            pltpu.async_copy(rows[b], out_hbm.at[base + v], wsem[b])
            if do_gather:
                bn = (b + _AHEAD) % _NBUF
                if do_wwait:
                    wait_write(bn)
                start_gather(v + _AHEAD, bn)

        start_gather(0, 0)
        start_gather(1, 1)

        for b in range(_NBUF):
            visit(b, b, do_wwait=(b >= _AHEAD), do_gather=True)

        def body(k, carry):
            for b in range(_NBUF):
                visit(_NBUF * k + b, b, do_wwait=True, do_gather=True)
            return carry

        lax.fori_loop(1, _ROWS_PER_W // _NBUF - 1, body, 0)

        last = _ROWS_PER_W - _NBUF
        for b in range(_NBUF):
            visit(last + b, b, do_wwait=(b < _AHEAD), do_gather=(b < _AHEAD))

        for b in range(_NBUF):
            wait_write(b)

    return emb_kernel


_EMB_KERNEL = _make_kernel()


@jax.jit
def kernel(x, emb_table):
    pos = jnp.asarray(_POS)
    return _EMB_KERNEL(x.astype(jnp.int32), emb_table, pos)


# speedup vs baseline: 1.5768x; 1.3261x over previous
import functools
import math

import numpy as np
import jax
import jax.numpy as jnp
from jax import lax
from jax.experimental import pallas as pl
from jax.experimental.pallas import tpu as pltpu
from jax.experimental.pallas import tpu_sc as plsc

_B, _S, _E = 4096, 200, 64
_NW = 32
_ROWS_PER_W = _B // _NW
_NBUF = 4
_AHEAD = 2


def _pos_encoding_np(seq_len=_S, emb_size=_E):
    position_idx = np.arange(0, seq_len, dtype=np.float32)[:, None]
    fill = position_idx * np.exp(
        -np.arange(0, emb_size, 2, dtype=np.float32) * math.log(10000.0) / emb_size)
    pos = np.zeros((seq_len, emb_size), dtype=np.float32)
    pos[:, 0::2] = np.sin(fill)
    pos[:, 1::2] = np.cos(fill)
    return pos


_POS = _pos_encoding_np()


def _make_kernel():
    mesh = plsc.VectorSubcoreMesh(core_axis_name="c", subcore_axis_name="s")

    row_buf = pltpu.VMEM((_S, _E), jnp.float32)
    scratch = (
        [pltpu.VMEM((_ROWS_PER_W, _S), jnp.int32)]
        + [pltpu.VMEM((_S, _E), jnp.float32)]
        + [row_buf] * _NBUF
        + [pltpu.SemaphoreType.DMA] * _NBUF
        + [pltpu.SemaphoreType.DMA] * _NBUF
    )

    @functools.partial(
        pl.kernel,
        out_type=jax.ShapeDtypeStruct((_B, _S, 2 * _E), jnp.float32),
        mesh=mesh,
        scratch_types=scratch,
        compiler_params=pltpu.CompilerParams(use_tc_tiling_on_sc=False),
    )
    def emb_kernel(x_hbm, table_hbm, pos_hbm, out_hbm, idx_all, pos_v,
                   r0, r1, r2, r3, g0, g1, g2, g3, w0, w1, w2, w3):
        rows = (r0, r1, r2, r3)
        gsem = (g0, g1, g2, g3)
        wsem = (w0, w1, w2, w3)

        wid = lax.axis_index("s") * 2 + lax.axis_index("c")
        base = wid * _ROWS_PER_W

        pltpu.sync_copy(pos_hbm, pos_v)
        pltpu.sync_copy(x_hbm.at[pl.ds(base, _ROWS_PER_W)], idx_all)

        def start_gather(v, b):
            pltpu.async_copy(table_hbm.at[idx_all.at[v]], rows[b], gsem[b])

        def wait_gather(b):
            pltpu.make_async_copy(table_hbm.at[idx_all.at[0]], rows[b],
                                  gsem[b]).wait()

        def wait_write(b):
            pltpu.make_async_copy(rows[b], out_hbm.at[base, :, pl.ds(0, _E)],
                                  wsem[b]).wait()

        def add_pos(b):
            buf = rows[b]

            @plsc.parallel_loop(0, _S, unroll=4)
            def _(r):
                for c in range(_E // 16):
                    sl = pl.ds(c * 16, 16)
                    buf[r, sl] = buf[r, sl] + pos_v[r, sl]

        def visit(v, b, do_wwait, do_gather):
            wait_gather(b)
            add_pos(b)
            pltpu.async_copy(rows[b], out_hbm.at[base + v, :, pl.ds(0, _E)],
                             wsem[b])
            if do_gather:
                bn = (b + _AHEAD) % _NBUF
                if do_wwait:
                    wait_write(bn)
                start_gather(v + _AHEAD, bn)

        start_gather(0, 0)
        start_gather(1, 1)

        for b in range(_NBUF):
            visit(b, b, do_wwait=(b >= _AHEAD), do_gather=True)

        def body(k, carry):
            for b in range(_NBUF):
                visit(_NBUF * k + b, b, do_wwait=True, do_gather=True)
            return carry

        lax.fori_loop(1, _ROWS_PER_W // _NBUF - 1, body, 0)

        last = _ROWS_PER_W - _NBUF
        for b in range(_NBUF):
            visit(last + b, b, do_wwait=(b < _AHEAD), do_gather=(b < _AHEAD))

        for b in range(_NBUF):
            wait_write(b)

    return emb_kernel


_EMB_KERNEL = _make_kernel()


@jax.jit
def kernel(x, emb_table):
    pos = jnp.asarray(_POS)
    out128 = _EMB_KERNEL(x.astype(jnp.int32), emb_table, pos)
    return out128[:, :, :_E]
